# baseline (speedup 1.0000x reference)
import functools

import jax
import jax.numpy as jnp
from jax import lax
from jax.experimental import pallas as pl
from jax.experimental.pallas import tpu as pltpu
from jax.experimental.pallas import tpu_sc as plsc

B = 8
NC = 2
NS = 16
NW = NC * NS
B_PER_CORE = B // NC
TILES_PER_B = NS // B_PER_CORE
LANES = 16
ROWS = 512
COLS = 512
ROWS_PER_TILE = ROWS // TILES_PER_B
CR = 16
CHUNK = CR * COLS
NCHUNK = ROWS_PER_TILE // CR
UNROLL = 4
VPC = CHUNK // (LANES * UNROLL)

_mesh = plsc.VectorSubcoreMesh(core_axis_name="c", subcore_axis_name="s")


@functools.partial(
    pl.kernel,
    mesh=_mesh,
    out_type=(
        jax.ShapeDtypeStruct((NW, 8, LANES), jnp.float32),
        jax.ShapeDtypeStruct((NC, LANES), jnp.float32),
    ),
    compiler_params=pltpu.CompilerParams(
        needs_layout_passes=False, use_tc_tiling_on_sc=True),
    scratch_types=[
        pltpu.VMEM((3, CR, COLS), jnp.float32),
        pltpu.VMEM((3, CR, COLS), jnp.float32),
        pltpu.VMEM((3, CR, COLS), jnp.int32),
        pltpu.VMEM((8, LANES), jnp.float32),
        pltpu.VMEM((NS, 8, LANES), jnp.float32),
        pltpu.VMEM((LANES,), jnp.float32),
        pltpu.SemaphoreType.DMA,
        pltpu.SemaphoreType.DMA,
        pltpu.SemaphoreType.DMA,
    ],
)
def _masked_loss(out_hbm, tgt_hbm, msk_hbm, part_hbm, loss_hbm, obuf, tbuf,
                 mbuf, pvec, pbuf, lbuf, sem0, sem1, sem2):
    c = lax.axis_index("c")
    s = lax.axis_index("s")
    b = c * B_PER_CORE + s // TILES_PER_B
    r_base = (s % TILES_PER_B) * ROWS_PER_TILE
    row = c * NS + s
    zero = jnp.zeros((LANES,), jnp.float32)
    one = jnp.ones((LANES,), jnp.float32)
    sems = (sem0, sem1, sem2)
    nbuf = 3
    lanes0 = lax.iota(jnp.int32, LANES)
    ohs = [jnp.where(lanes0 == i, one, zero) for i in range(1, 5)]
    dn0 = lax.GatherDimensionNumbers(
        offset_dims=(), collapsed_slice_dims=(0,), start_index_map=(0,))

    def onehot(tbl, mm):
        return lax.gather(tbl, mm, dn0, slice_sizes=(1,),
                          mode=lax.GatherScatterMode.PROMISE_IN_BOUNDS)

    def start_fetch(ci):
        p = ci % nbuf
        sl = pl.ds(r_base + ci * CR, CR)
        return [
            pltpu.async_copy(out_hbm.at[b, sl, :], obuf.at[p], sems[p]),
            pltpu.async_copy(tgt_hbm.at[b, sl, :], tbuf.at[p], sems[p]),
            pltpu.async_copy(msk_hbm.at[b, sl, :], mbuf.at[p], sems[p]),
        ]

    starts = {0: start_fetch(0), 1: start_fetch(1)}
    acc = (zero,) * 8
    for ci in range(NCHUNK):
        p = ci % nbuf
        for cp in starts.pop(ci):
            cp.wait()
        if ci + 2 < NCHUNK:
            starts[ci + 2] = start_fetch(ci + 2)

        def vec_body(j, carry, p=p):
            a1, a2, a3, a4, c1, c2, c3, c4 = carry
            ipr = COLS // (LANES * UNROLL)
            i = j // ipr
            c0 = (j % ipr) * (LANES * UNROLL)
            for k in range(UNROLL):
                sl = pl.ds(c0 + k * LANES, LANES)
                o = obuf[p, i, sl]
                t = tbuf[p, i, sl]
                m = mbuf[p, i, sl]
                d = o - t
                d2 = d * d
                mm = jnp.reshape(m, (LANES, 1))
                f1 = onehot(ohs[0], mm)
                f2 = onehot(ohs[1], mm)
                f3 = onehot(ohs[2], mm)
                f4 = onehot(ohs[3], mm)
                a1 = a1 + d2 * f1
                a2 = a2 + d2 * f2
                a3 = a3 + d2 * f3
                a4 = a4 + d2 * f4
                c1 = c1 + f1
                c2 = c2 + f2
                c3 = c3 + f3
                c4 = c4 + f4
            return (a1, a2, a3, a4, c1, c2, c3, c4)

        acc = lax.fori_loop(0, VPC, vec_body, acc)

    for k in range(8):
        pvec[k, :] = acc[k]
    pltpu.sync_copy(pvec, part_hbm.at[row])
    plsc.subcore_barrier()

    @pl.when(s == 0)
    def _():
        pltpu.sync_copy(part_hbm.at[pl.ds(c * NS, NS)], pbuf)
        lanes = lax.iota(jnp.int32, LANES)
        perms = [jnp.reshape(jnp.bitwise_xor(lanes, d), (LANES, 1))
                 for d in (1, 2, 4, 8)]
        dn = lax.GatherDimensionNumbers(
            offset_dims=(), collapsed_slice_dims=(0,), start_index_map=(0,))

        def lane_sum(v):
            for pm in perms:
                v = v + lax.gather(v, pm, dn, slice_sizes=(1,),
                                   mode=lax.GatherScatterMode.PROMISE_IN_BOUNDS)
            return v

        lossv = zero
        for bb in range(B_PER_CORE):
            t0 = TILES_PER_B * bb
            for i in range(4):
                v = (pbuf[t0 + 0, i, :] + pbuf[t0 + 1, i, :]
                     + pbuf[t0 + 2, i, :] + pbuf[t0 + 3, i, :])
                cc = (pbuf[t0 + 0, i + 4, :] + pbuf[t0 + 1, i + 4, :]
                      + pbuf[t0 + 2, i + 4, :] + pbuf[t0 + 3, i + 4, :])
                sv = lane_sum(v)
                cv = lane_sum(cc)
                contrib = jnp.where(cv > 0.0, sv / jnp.maximum(cv, 1.0), zero)
                lossv = lossv + contrib
        lbuf[...] = lossv * jnp.float32(1.0 / B)
        pltpu.sync_copy(lbuf, loss_hbm.at[c])


def kernel(output, target, mask):
    _, loss = _masked_loss(output, target, mask)
    return loss[0, 0] + loss[1, 0]


# speedup vs baseline: 1.2287x; 1.2287x over previous
import functools

import jax
import jax.numpy as jnp
from jax import lax
from jax.experimental import pallas as pl
from jax.experimental.pallas import tpu as pltpu
from jax.experimental.pallas import tpu_sc as plsc

B = 8
B_SC = 4
B_TC = B - B_SC
NC = 2
NS = 16
B_PER_CORE = B_SC // NC
TILES_PER_B = NS // B_PER_CORE
LANES = 16
ROWS = 512
COLS = 512
ROWS_PER_TILE = ROWS // TILES_PER_B
CR = 16
CHUNK = CR * COLS
NCHUNK = ROWS_PER_TILE // CR
UNROLL = 4
VPC = CHUNK // (LANES * UNROLL)

TC_BR = 128
TC_NRB = ROWS // TC_BR

_mesh = plsc.VectorSubcoreMesh(core_axis_name="c", subcore_axis_name="s")


@functools.partial(
    pl.kernel,
    mesh=_mesh,
    out_type=(
        jax.ShapeDtypeStruct((NW := NC * NS, 8, LANES), jnp.float32),
        jax.ShapeDtypeStruct((NC, LANES), jnp.float32),
    ),
    compiler_params=pltpu.CompilerParams(
        needs_layout_passes=False, use_tc_tiling_on_sc=True),
    scratch_types=[
        pltpu.VMEM((2, CR, COLS), jnp.float32),
        pltpu.VMEM((2, CR, COLS), jnp.float32),
        pltpu.VMEM((2, CR, COLS), jnp.int32),
        pltpu.VMEM((8, LANES), jnp.float32),
        pltpu.VMEM((NS, 8, LANES), jnp.float32),
        pltpu.VMEM((LANES,), jnp.float32),
        pltpu.SemaphoreType.DMA,
        pltpu.SemaphoreType.DMA,
    ],
)
def _masked_loss_sc(out_hbm, tgt_hbm, msk_hbm, part_hbm, loss_hbm, obuf, tbuf,
                    mbuf, pvec, pbuf, lbuf, sem0, sem1):
    c = lax.axis_index("c")
    s = lax.axis_index("s")
    b = c * B_PER_CORE + s // TILES_PER_B
    r_base = (s % TILES_PER_B) * ROWS_PER_TILE
    row = c * NS + s
    zero = jnp.zeros((LANES,), jnp.float32)
    one = jnp.ones((LANES,), jnp.float32)
    sems = (sem0, sem1)
    nbuf = 2
    lanes0 = lax.iota(jnp.int32, LANES)
    ohs = [jnp.where(lanes0 == i, one, zero) for i in range(1, 5)]
    dn0 = lax.GatherDimensionNumbers(
        offset_dims=(), collapsed_slice_dims=(0,), start_index_map=(0,))

    def onehot(tbl, mm):
        return lax.gather(tbl, mm, dn0, slice_sizes=(1,),
                          mode=lax.GatherScatterMode.PROMISE_IN_BOUNDS)

    def start_fetch(ci):
        p = ci % nbuf
        sl = pl.ds(r_base + ci * CR, CR)
        return [
            pltpu.async_copy(out_hbm.at[b, sl, :], obuf.at[p], sems[p]),
            pltpu.async_copy(tgt_hbm.at[b, sl, :], tbuf.at[p], sems[p]),
            pltpu.async_copy(msk_hbm.at[b, sl, :], mbuf.at[p], sems[p]),
        ]

    inflight = start_fetch(0)
    acc = (zero,) * 8
    for ci in range(NCHUNK):
        p = ci % nbuf
        for cp in inflight:
            cp.wait()
        if ci + 1 < NCHUNK:
            inflight = start_fetch(ci + 1)

        def vec_body(j, carry, p=p):
            a1, a2, a3, a4, c1, c2, c3, c4 = carry
            ipr = COLS // (LANES * UNROLL)
            i = j // ipr
            c0 = (j % ipr) * (LANES * UNROLL)
            for k in range(UNROLL):
                sl = pl.ds(c0 + k * LANES, LANES)
                o = obuf[p, i, sl]
                t = tbuf[p, i, sl]
                m = mbuf[p, i, sl]
                d = o - t
                d2 = d * d
                mm = jnp.reshape(m, (LANES, 1))
                f1 = onehot(ohs[0], mm)
                f2 = onehot(ohs[1], mm)
                f3 = onehot(ohs[2], mm)
                f4 = onehot(ohs[3], mm)
                a1 = a1 + d2 * f1
                a2 = a2 + d2 * f2
                a3 = a3 + d2 * f3
                a4 = a4 + d2 * f4
                c1 = c1 + f1
                c2 = c2 + f2
                c3 = c3 + f3
                c4 = c4 + f4
            return (a1, a2, a3, a4, c1, c2, c3, c4)

        acc = lax.fori_loop(0, VPC, vec_body, acc)

    for k in range(8):
        pvec[k, :] = acc[k]
    pltpu.sync_copy(pvec, part_hbm.at[row])
    plsc.subcore_barrier()

    @pl.when(s == 0)
    def _():
        pltpu.sync_copy(part_hbm.at[pl.ds(c * NS, NS)], pbuf)
        lanes = lax.iota(jnp.int32, LANES)
        perms = [jnp.reshape(jnp.bitwise_xor(lanes, d), (LANES, 1))
                 for d in (1, 2, 4, 8)]

        def lane_sum(v):
            for pm in perms:
                v = v + lax.gather(v, pm, dn0, slice_sizes=(1,),
                                   mode=lax.GatherScatterMode.PROMISE_IN_BOUNDS)
            return v

        lossv = zero
        for bb in range(B_PER_CORE):
            t0 = TILES_PER_B * bb
            for i in range(4):
                v = zero
                cc = zero
                for tt in range(TILES_PER_B):
                    v = v + pbuf[t0 + tt, i, :]
                    cc = cc + pbuf[t0 + tt, i + 4, :]
                sv = lane_sum(v)
                cv = lane_sum(cc)
                contrib = jnp.where(cv > 0.0, sv / jnp.maximum(cv, 1.0), zero)
                lossv = lossv + contrib
        lbuf[...] = lossv * jnp.float32(1.0 / B)
        pltpu.sync_copy(lbuf, loss_hbm.at[c])


def _tc_body(o_ref, t_ref, m_ref, loss_ref, sums_ref):
    r = pl.program_id(1)
    bt = pl.program_id(0)
    d = o_ref[0] - t_ref[0]
    d2 = d * d
    m = m_ref[0]
    lane = lax.broadcasted_iota(jnp.int32, (8, 128), 1)
    rowi = lax.broadcasted_iota(jnp.int32, (8, 128), 0)
    vec = jnp.zeros((8, 128), jnp.float32)
    for i in range(1, 5):
        sel = m == i
        s_i = jnp.sum(jnp.where(sel, d2, 0.0))
        c_i = jnp.sum(jnp.where(sel, 1.0, 0.0))
        vec = vec + jnp.where(lane == i, s_i, 0.0)
        vec = vec + jnp.where(lane == i + 8, c_i, 0.0)
    vec = jnp.where(rowi == bt, vec, 0.0)

    first = jnp.logical_and(bt == 0, r == 0)

    @pl.when(first)
    def _():
        sums_ref[...] = vec

    @pl.when(jnp.logical_not(first))
    def _():
        sums_ref[...] = sums_ref[...] + vec

    @pl.when(jnp.logical_and(bt == B_TC - 1, r == TC_NRB - 1))
    def _():
        tot = sums_ref[...]
        contrib = jnp.float32(0.0)
        for bb in range(B_TC):
            for i in range(1, 5):
                s_i = tot[bb, i]
                c_i = tot[bb, i + 8]
                contrib = contrib + jnp.where(
                    c_i > 0.0, s_i / jnp.maximum(c_i, 1.0), 0.0)
        lane1 = lax.broadcasted_iota(jnp.int32, (1, 128), 1)
        loss_ref[...] = jnp.where(
            lane1 == 0, contrib * jnp.float32(1.0 / B), 0.0)


_tc_loss = pl.pallas_call(
    _tc_body,
    grid=(B_TC, TC_NRB),
    in_specs=[
        pl.BlockSpec((1, TC_BR, COLS), lambda bt, r: (B_SC + bt, r, 0)),
        pl.BlockSpec((1, TC_BR, COLS), lambda bt, r: (B_SC + bt, r, 0)),
        pl.BlockSpec((1, TC_BR, COLS), lambda bt, r: (B_SC + bt, r, 0)),
    ],
    out_specs=pl.BlockSpec((1, 128), lambda bt, r: (0, 0)),
    out_shape=jax.ShapeDtypeStruct((1, 128), jnp.float32),
    scratch_shapes=[pltpu.VMEM((8, 128), jnp.float32)],
)


def kernel(output, target, mask):
    _, sc_loss = _masked_loss_sc(output, target, mask)
    tc_loss = _tc_loss(output, target, mask)
    return sc_loss[0, 0] + sc_loss[1, 0] + tc_loss[0, 0]
